# baseline (speedup 1.0000x reference)
import functools

import jax
import jax.numpy as jnp
from jax import lax
from jax.experimental import pallas as pl
from jax.experimental.pallas import tpu as pltpu
from jax.experimental.pallas import tpu_sc as plsc

_D = 1024
_B = 4 * 4096
_NC = 2
_NS = 16
_NW = _NC * _NS
_BPW = _B // _NW
_CH = 32
_NCHUNK = _BPW // _CH

_VROWS = 4 * 32 * 256
_VBLK = 2048


_VM_SHAPE = (4, 32, 256)
_BM_SHAPE = (4, 4096)


@functools.partial(
    pl.kernel,
    out_type=(
        jax.ShapeDtypeStruct((_B, _D), jnp.float32),
        jax.ShapeDtypeStruct(_VM_SHAPE, jnp.bool_),
        jax.ShapeDtypeStruct(_BM_SHAPE, jnp.bool_),
    ),
    mesh=plsc.VectorSubcoreMesh(
        core_axis_name="c", subcore_axis_name="s",
        num_cores=_NC, num_subcores=_NS,
    ),
    scratch_types=[
        pltpu.VMEM((_BPW,), jnp.int32),
        pltpu.VMEM((2, _CH, _D), jnp.float32),
        pltpu.VMEM(_VM_SHAPE, jnp.bool_),
        pltpu.VMEM(_BM_SHAPE, jnp.bool_),
        pltpu.SemaphoreType.DMA,
        pltpu.SemaphoreType.DMA,
        pltpu.SemaphoreType.DMA,
        pltpu.SemaphoreType.DMA,
    ],
)
def _embed_gather(table_hbm, idx_hbm, vm_hbm, bm_hbm,
                  out_hbm, m1_hbm, m2_hbm,
                  idx_v, bufs, vmbuf, bmbuf, gsem0, gsem1, ssem0, ssem1):
    wid = lax.axis_index("s") * _NC + lax.axis_index("c")
    base = wid * _BPW
    pltpu.sync_copy(idx_hbm.at[pl.ds(base, _BPW)], idx_v)

    @pl.when(wid == 0)
    def _copy_masks():
        pltpu.sync_copy(vm_hbm, vmbuf)
        pltpu.sync_copy(vmbuf, m1_hbm)
        pltpu.sync_copy(bm_hbm, bmbuf)
        pltpu.sync_copy(bmbuf, m2_hbm)

    gsems = (gsem0, gsem1)
    ssems = (ssem0, ssem1)
    gathers = [None, None]
    scatters = [None, None]

    def start_gather(c):
        b = c % 2
        gathers[b] = pltpu.async_copy(
            table_hbm.at[idx_v.at[pl.ds(c * _CH, _CH)]],
            bufs.at[b],
            gsems[b],
        )

    start_gather(0)
    for c in range(_NCHUNK):
        b = c % 2
        if c + 1 < _NCHUNK:
            if scatters[1 - b] is not None:
                scatters[1 - b].wait()
            start_gather(c + 1)
        gathers[b].wait()
        scatters[b] = pltpu.async_copy(
            bufs.at[b],
            out_hbm.at[pl.ds(base + c * _CH, _CH)],
            ssems[b],
        )
    scatters[0].wait()
    scatters[1].wait()


def _dup_copy_body(x_ref, o1_ref, o2_ref):
    v = x_ref[...]
    o1_ref[...] = v
    o2_ref[...] = v


_dup_copy = pl.pallas_call(
    _dup_copy_body,
    grid=(_VROWS // _VBLK,),
    in_specs=[pl.BlockSpec((_VBLK, _D), lambda i: (i, 0))],
    out_specs=[
        pl.BlockSpec((_VBLK, _D), lambda i: (i, 0)),
        pl.BlockSpec((_VBLK, _D), lambda i: (i, 0)),
    ],
    out_shape=(
        jax.ShapeDtypeStruct((_VROWS, _D), jnp.float32),
        jax.ShapeDtypeStruct((_VROWS, _D), jnp.float32),
    ),
)


def kernel(vision_feats, text_tokens, embed_table,
           vision_xattn_mask, buffer_xattn_mask):
    idx = text_tokens.reshape(-1)
    emb, m1, m2 = _embed_gather(
        embed_table, idx, vision_xattn_mask, buffer_xattn_mask)
    embedded_text = emb.reshape(
        text_tokens.shape[0], text_tokens.shape[1], _D)
    v1, v2 = _dup_copy(vision_feats.reshape(_VROWS, _D))
    vshape = vision_feats.shape
    return (
        embedded_text,
        v1.reshape(vshape),
        v2.reshape(vshape),
        m1,
        m2,
    )


# speedup vs baseline: 1.0296x; 1.0296x over previous
import functools

import jax
import jax.numpy as jnp
from jax import lax
from jax.experimental import pallas as pl
from jax.experimental.pallas import tpu as pltpu
from jax.experimental.pallas import tpu_sc as plsc

_D = 1024
_B = 4 * 4096
_NC = 2
_NS = 16
_NW = _NC * _NS
_BPW = _B // _NW
_CH = 32
_NCHUNK = _BPW // _CH

_VROWS = 4 * 32 * 256
_VBLK = 2048


@functools.partial(
    pl.kernel,
    out_type=jax.ShapeDtypeStruct((_B, _D), jnp.float32),
    mesh=plsc.VectorSubcoreMesh(
        core_axis_name="c", subcore_axis_name="s",
        num_cores=_NC, num_subcores=_NS,
    ),
    scratch_types=[
        pltpu.VMEM((_BPW,), jnp.int32),
        pltpu.VMEM((2, _CH, _D), jnp.float32),
        pltpu.SemaphoreType.DMA,
        pltpu.SemaphoreType.DMA,
        pltpu.SemaphoreType.DMA,
        pltpu.SemaphoreType.DMA,
    ],
)
def _embed_gather(table_hbm, idx_hbm, out_hbm, idx_v, bufs,
                  gsem0, gsem1, ssem0, ssem1):
    wid = lax.axis_index("s") * _NC + lax.axis_index("c")
    base = wid * _BPW
    pltpu.sync_copy(
        idx_hbm.at[wid // 8, pl.ds((wid % 8) * _BPW, _BPW)], idx_v)

    gsems = (gsem0, gsem1)
    ssems = (ssem0, ssem1)
    gathers = [None, None]
    scatters = [None, None]

    def start_gather(c):
        b = c % 2
        gathers[b] = pltpu.async_copy(
            table_hbm.at[idx_v.at[pl.ds(c * _CH, _CH)]],
            bufs.at[b],
            gsems[b],
        )

    start_gather(0)
    for c in range(_NCHUNK):
        b = c % 2
        if c + 1 < _NCHUNK:
            if scatters[1 - b] is not None:
                scatters[1 - b].wait()
            start_gather(c + 1)
        gathers[b].wait()
        scatters[b] = pltpu.async_copy(
            bufs.at[b],
            out_hbm.at[pl.ds(base + c * _CH, _CH)],
            ssems[b],
        )
    scatters[0].wait()
    scatters[1].wait()


def _dup_copy_body(x_ref, o1_ref, o2_ref):
    v = x_ref[...]
    o1_ref[...] = v
    o2_ref[...] = v


_dup_copy = pl.pallas_call(
    _dup_copy_body,
    grid=(_VROWS // _VBLK,),
    in_specs=[pl.BlockSpec((_VBLK, _D), lambda i: (i, 0))],
    out_specs=[
        pl.BlockSpec((_VBLK, _D), lambda i: (i, 0)),
        pl.BlockSpec((_VBLK, _D), lambda i: (i, 0)),
    ],
    out_shape=(
        jax.ShapeDtypeStruct((_VROWS, _D), jnp.float32),
        jax.ShapeDtypeStruct((_VROWS, _D), jnp.float32),
    ),
)


def kernel(vision_feats, text_tokens, embed_table,
           vision_xattn_mask, buffer_xattn_mask):
    emb = _embed_gather(embed_table, text_tokens)
    embedded_text = emb.reshape(
        text_tokens.shape[0], text_tokens.shape[1], _D)
    v1, v2 = _dup_copy(vision_feats.reshape(_VROWS, _D))
    vshape = vision_feats.shape
    return (
        embedded_text,
        v1.reshape(vshape),
        v2.reshape(vshape),
        vision_xattn_mask,
        buffer_xattn_mask,
    )
